# Initial kernel scaffold; baseline (speedup 1.0000x reference)
import jax
import jax.numpy as jnp
from jax.experimental import pallas as pl


def kernel(features, in_points, out_points, node_weights, eval_indices, W0, W1, W2, W3, W4, W5):
    raise NotImplementedError("write your pallas kernel here")


# speedup vs baseline: 18.3600x; 18.3600x over previous
import functools

import jax
import jax.numpy as jnp
from jax import lax
from jax.experimental import pallas as pl
from jax.experimental.pallas import tpu as pltpu
from jax.experimental.pallas import tpu_sc as plsc

N_PTS = 50000
E = 800000
NC, NS = 2, 16
NW = NC * NS
E_PAD = 819200
ROWS_W = 200
E_W = ROWS_W * 128
G = 8
N_GROUPS = ROWS_W // G
CH = G * 128
N_ACC = 50048
ROWS_T = N_PTS // NS

_SC = (0.9999999778617678, -0.16666645303901012, 0.008333121801711259,
       -0.00019833888177304604, 2.7436987965212285e-06,
       -2.404730690096224e-08, 1.1825182499869413e-10)


def _sin(x):
    z = x * x
    p = jnp.float32(_SC[6])
    for c in (_SC[5], _SC[4], _SC[3], _SC[2], _SC[1], _SC[0]):
        p = p * z + jnp.float32(c)
    return x * p


_PBLK = 2000


def _prep_body(f_ref, w_ref, op_ref, ip_ref, o_ref, pts_ref):
    o_ref[...] = f_ref[...] * w_ref[...]
    zpad = jnp.zeros((_PBLK, 12), jnp.float32)
    pts_ref[...] = jnp.concatenate([op_ref[...], ip_ref[...], zpad], axis=1)


def _prep(feat_rows, nw_col, out_points, in_points):
    return pl.pallas_call(
        _prep_body,
        grid=(N_PTS // _PBLK,),
        in_specs=[
            pl.BlockSpec((_PBLK, 16), lambda i: (i, 0)),
            pl.BlockSpec((_PBLK, 1), lambda i: (i, 0)),
            pl.BlockSpec((_PBLK, 2), lambda i: (i, 0)),
            pl.BlockSpec((_PBLK, 2), lambda i: (i, 0)),
        ],
        out_specs=(pl.BlockSpec((_PBLK, 16), lambda i: (i, 0)),
                   pl.BlockSpec((_PBLK, 16), lambda i: (i, 0))),
        out_shape=(jax.ShapeDtypeStruct((N_PTS, 16), jnp.float32),
                   jax.ShapeDtypeStruct((N_PTS, 16), jnp.float32)),
    )(feat_rows, nw_col, out_points, in_points)


@functools.cache
def _make_gather():
    mesh = plsc.VectorSubcoreMesh(core_axis_name="c", subcore_axis_name="s",
                                  num_cores=NC, num_subcores=NS)

    @functools.partial(
        pl.kernel,
        out_type=(
            jax.ShapeDtypeStruct((E_PAD, 16), jnp.float32),
            jax.ShapeDtypeStruct((E_PAD, 2), jnp.float32),
            jax.ShapeDtypeStruct((E_PAD, 2), jnp.float32),
        ),
        mesh=mesh,
        scratch_types=[
            pltpu.VMEM((G, 128), jnp.int32),
            pltpu.VMEM((G, 128), jnp.int32),
            pltpu.VMEM((CH, 16), jnp.float32),
            pltpu.VMEM((CH, 16), jnp.float32),
            pltpu.VMEM((CH, 16), jnp.float32),
            pltpu.SemaphoreType.DMA,
        ],
        compiler_params=pltpu.CompilerParams(use_tc_tiling_on_sc=False),
    )
    def gather_k(idxo_hbm, idxi_hbm, wft_hbm, pts_hbm,
                 wf_hbm, po_hbm, pi_hbm,
                 idxo_v, idxi_v, wf_v, g1_v, g2_v, sem):
        wid = lax.axis_index("s") * NC + lax.axis_index("c")

        def group(g, carry):
            base = wid * E_W + g * CH
            pltpu.sync_copy(idxo_hbm.at[wid, pl.ds(g * G, G)], idxo_v)
            pltpu.sync_copy(idxi_hbm.at[wid, pl.ds(g * G, G)], idxi_v)
            descs = []
            for j in range(G):
                descs.append(pltpu.async_copy(
                    wft_hbm.at[idxi_v.at[j]], wf_v.at[pl.ds(j * 128, 128)], sem))
                descs.append(pltpu.async_copy(
                    pts_hbm.at[idxo_v.at[j]], g1_v.at[pl.ds(j * 128, 128)], sem))
                descs.append(pltpu.async_copy(
                    pts_hbm.at[idxi_v.at[j]], g2_v.at[pl.ds(j * 128, 128)], sem))
            for d in descs:
                d.wait()
            pltpu.sync_copy(wf_v, wf_hbm.at[pl.ds(base, CH)])
            pltpu.sync_copy(g1_v.at[:, pl.ds(0, 2)], po_hbm.at[pl.ds(base, CH)])
            pltpu.sync_copy(g2_v.at[:, pl.ds(2, 2)], pi_hbm.at[pl.ds(base, CH)])
            return carry

        lax.fori_loop(0, N_GROUPS, group, 0)

    return gather_k


_EBLK = 3200


def _mlp_body(po_ref, pi_ref, wf_ref, w0_ref, w1_ref, w2_ref, w3_ref, w4_ref,
              w5_ref, eye_ref, out_ref):
    locs = po_ref[...] - pi_ref[...]
    h = lax.dot_general(w0_ref[...], locs, (((1,), (1,)), ((), ())),
                        preferred_element_type=jnp.float32)
    h = _sin(h)
    for wref in (w1_ref, w2_ref, w3_ref, w4_ref):
        h = _sin(lax.dot_general(wref[...], h, (((1,), (0,)), ((), ())),
                                 preferred_element_type=jnp.float32))
    h5 = lax.dot_general(w5_ref[...], h, (((1,), (0,)), ((), ())),
                         preferred_element_type=jnp.float32)
    wft = lax.dot_general(eye_ref[...], wf_ref[...], (((1,), (1,)), ((), ())),
                          preferred_element_type=jnp.float32,
                          precision=lax.Precision.HIGHEST)
    wf4 = wft.reshape(4, 4, _EBLK)
    h4 = h5.reshape(4, 4, _EBLK)
    valt = jnp.sum(wf4[:, :, None, :] * h4[None, :, :, :], axis=1)
    valt = valt.reshape(16, _EBLK)
    vals = lax.dot_general(valt, eye_ref[...], (((0,), (0,)), ((), ())),
                           preferred_element_type=jnp.float32,
                           precision=lax.Precision.HIGHEST)
    out_ref[...] = vals


def _mlp(po, pi, wf, ws, eye):
    wspecs = [pl.BlockSpec(w.shape, lambda i: (0, 0)) for w in ws]
    return pl.pallas_call(
        _mlp_body,
        grid=(E_PAD // _EBLK,),
        in_specs=[
            pl.BlockSpec((_EBLK, 2), lambda i: (i, 0)),
            pl.BlockSpec((_EBLK, 2), lambda i: (i, 0)),
            pl.BlockSpec((_EBLK, 16), lambda i: (i, 0)),
            *wspecs,
            pl.BlockSpec((16, 16), lambda i: (0, 0)),
        ],
        out_specs=pl.BlockSpec((_EBLK, 16), lambda i: (i, 0)),
        out_shape=jax.ShapeDtypeStruct((E_PAD, 16), jnp.float32),
    )(po, pi, wf, *ws, eye)


@functools.cache
def _make_scatter():
    mesh = plsc.VectorSubcoreMesh(core_axis_name="c", subcore_axis_name="s",
                                  num_cores=NC, num_subcores=NS)

    @functools.partial(
        pl.kernel,
        out_type=jax.ShapeDtypeStruct((NC, N_PTS, 16), jnp.float32),
        mesh=mesh,
        scratch_types=[
            pltpu.VMEM((G, 128), jnp.int32),
            pltpu.VMEM((CH, 16), jnp.float32),
            pltpu.VMEM((125, 16), jnp.float32),
            pltpu.VMEM_SHARED((N_ACC, 16), jnp.float32),
            pltpu.SemaphoreType.DMA,
        ],
        compiler_params=pltpu.CompilerParams(use_tc_tiling_on_sc=False),
    )
    def scatter_k(idxo_hbm, val_hbm, out_hbm, idxo_v, val_v, zb_v, acc_sh, sem):
        c = lax.axis_index("c")
        s = lax.axis_index("s")
        wid = s * NC + c

        zero16 = jnp.zeros((16,), jnp.float32)

        def zrow(i, carry):
            zb_v[i] = zero16
            return carry

        lax.fori_loop(0, 125, zrow, 0)

        def zacc(i, carry):
            pltpu.sync_copy(zb_v, acc_sh.at[pl.ds(s * ROWS_T + i * 125, 125)])
            return carry

        lax.fori_loop(0, 25, zacc, 0)

        @pl.when(s == 0)
        def _():
            pltpu.sync_copy(zb_v, acc_sh.at[pl.ds(N_ACC - 125, 125)])

        plsc.subcore_barrier()

        def group(g, carry):
            base = wid * E_W + g * CH
            pltpu.sync_copy(idxo_hbm.at[wid, pl.ds(g * G, G)], idxo_v)
            pltpu.sync_copy(val_hbm.at[pl.ds(base, CH)], val_v)
            for j in range(G):
                pltpu.sync_copy(val_v.at[pl.ds(j * 128, 128)],
                                acc_sh.at[idxo_v.at[j]], add=True)
            return carry

        lax.fori_loop(0, N_GROUPS, group, 0)

        plsc.subcore_barrier()

        pltpu.sync_copy(acc_sh.at[pl.ds(s * ROWS_T, ROWS_T)],
                        out_hbm.at[c, pl.ds(s * ROWS_T, ROWS_T)])

    return scatter_k


_FBLK = 2000


def _final_body(p_ref, eye_ref, o_ref):
    psum = p_ref[0] + p_ref[1]
    o_ref[...] = lax.dot_general(eye_ref[...], psum, (((1,), (1,)), ((), ())),
                                 preferred_element_type=jnp.float32,
                                 precision=lax.Precision.HIGHEST)


def _final(partials, eye):
    return pl.pallas_call(
        _final_body,
        out_shape=jax.ShapeDtypeStruct((16, N_PTS), jnp.float32),
    )(partials, eye)


def kernel(features, in_points, out_points, node_weights, eval_indices,
           W0, W1, W2, W3, W4, W5):
    n_extra = E_PAD - E
    pad_src = (jnp.arange(n_extra, dtype=jnp.int32) * 37) % N_PTS
    pad_dst = N_PTS + (jnp.arange(n_extra, dtype=jnp.int32) % (N_ACC - N_PTS))
    idx_out = jnp.concatenate([eval_indices[:, 0], pad_dst])
    idx_in = jnp.concatenate([eval_indices[:, 1], pad_src])
    idxo3 = idx_out.reshape(NW, ROWS_W, 128)
    idxi3 = idx_in.reshape(NW, ROWS_W, 128)

    feat_rows = features.reshape(16, N_PTS).T
    nw_col = node_weights.reshape(N_PTS, 1)
    eye = jnp.eye(16, dtype=jnp.float32)

    wft, pts = _prep(feat_rows, nw_col, out_points, in_points)
    wf, po, pi = _make_gather()(idxo3, idxi3, wft, pts)
    vals = _mlp(po, pi, wf, (W0, W1, W2, W3, W4, W5), eye)
    partials = _make_scatter()(idxo3, vals)
    out16 = _final(partials, eye)
    return out16.reshape(4, 4, N_PTS)


# speedup vs baseline: 53.5343x; 2.9158x over previous
import functools

import jax
import jax.numpy as jnp
from jax import lax
from jax.experimental import pallas as pl
from jax.experimental.pallas import tpu as pltpu
from jax.experimental.pallas import tpu_sc as plsc

N_PTS = 50000
E = 800000
NC, NS = 2, 16
NW = NC * NS
E_PAD = 819200
ROWS_W = 200
E_W = ROWS_W * 128
G = 8
N_GROUPS = ROWS_W // G
CH = G * 128
N_ACC = 50048
ROWS_T = N_PTS // NS

_SC = (0.9999999778617678, -0.16666645303901012, 0.008333121801711259,
       -0.00019833888177304604, 2.7436987965212285e-06,
       -2.404730690096224e-08, 1.1825182499869413e-10)


def _sin(x):
    z = x * x
    p = jnp.float32(_SC[6])
    for c in (_SC[5], _SC[4], _SC[3], _SC[2], _SC[1], _SC[0]):
        p = p * z + jnp.float32(c)
    return x * p


N_PLANE = 50048
ROWS_P = N_PLANE // NS
@functools.cache
def _make_gather():
    mesh = plsc.VectorSubcoreMesh(core_axis_name="c", subcore_axis_name="s",
                                  num_cores=NC, num_subcores=NS)

    @functools.partial(
        pl.kernel,
        out_type=(
            jax.ShapeDtypeStruct((E_PAD, 16), jnp.float32),
            jax.ShapeDtypeStruct((3, E_PAD), jnp.float32),
        ),
        mesh=mesh,
        scratch_types=[
            pltpu.VMEM((G, 128), jnp.int32),
            pltpu.VMEM((G, 128), jnp.int32),
            pltpu.VMEM((CH, 16), jnp.float32),
            pltpu.VMEM((CH,), jnp.float32),
            pltpu.VMEM((CH,), jnp.float32),
            pltpu.VMEM((CH,), jnp.float32),
            pltpu.VMEM((CH,), jnp.float32),
            pltpu.VMEM((CH,), jnp.float32),
            pltpu.VMEM_SHARED((N_PTS, 16), jnp.float32),
            pltpu.VMEM_SHARED((5, N_PLANE), jnp.float32),
            pltpu.SemaphoreType.DMA,
        ],
        compiler_params=pltpu.CompilerParams(use_tc_tiling_on_sc=False),
    )
    def gather_k(idxo_hbm, idxi_hbm, ft_hbm, pln_hbm,
                 wf_hbm, lw_hbm,
                 idxo_v, idxi_v, wf_v, ox_v, oy_v, ix_v, iy_v, w_v,
                 ft_sh, pln_sh, sem):
        s = lax.axis_index("s")
        wid = s * NC + lax.axis_index("c")

        descs = [pltpu.async_copy(ft_hbm.at[pl.ds(s * ROWS_T, ROWS_T)],
                                  ft_sh.at[pl.ds(s * ROWS_T, ROWS_T)], sem)]
        for p in range(5):
            descs.append(pltpu.async_copy(
                pln_hbm.at[p, pl.ds(s * ROWS_P, ROWS_P)],
                pln_sh.at[p, pl.ds(s * ROWS_P, ROWS_P)], sem))
        for d in descs:
            d.wait()
        plsc.subcore_barrier()

        def group(g, carry):
            base = wid * E_W + g * CH
            pltpu.sync_copy(idxo_hbm.at[wid, pl.ds(g * G, G)], idxo_v)
            pltpu.sync_copy(idxi_hbm.at[wid, pl.ds(g * G, G)], idxi_v)
            descs = []
            for j in range(G):
                dst = pl.ds(j * 128, 128)
                descs.append(pltpu.async_copy(
                    ft_sh.at[idxi_v.at[j]], wf_v.at[dst], sem))
                descs.append(pltpu.async_copy(
                    pln_sh.at[0].at[idxo_v.at[j]], ox_v.at[dst], sem))
                descs.append(pltpu.async_copy(
                    pln_sh.at[1].at[idxo_v.at[j]], oy_v.at[dst], sem))
                descs.append(pltpu.async_copy(
                    pln_sh.at[2].at[idxi_v.at[j]], ix_v.at[dst], sem))
                descs.append(pltpu.async_copy(
                    pln_sh.at[3].at[idxi_v.at[j]], iy_v.at[dst], sem))
                descs.append(pltpu.async_copy(
                    pln_sh.at[4].at[idxi_v.at[j]], w_v.at[dst], sem))
            for d in descs:
                d.wait()
            pltpu.sync_copy(wf_v, wf_hbm.at[pl.ds(base, CH)])
            for q in range(CH // 16):
                sl = pl.ds(q * 16, 16)
                ox_v[sl] = ox_v[sl] - ix_v[sl]
                oy_v[sl] = oy_v[sl] - iy_v[sl]
            pltpu.sync_copy(ox_v, lw_hbm.at[0, pl.ds(base, CH)])
            pltpu.sync_copy(oy_v, lw_hbm.at[1, pl.ds(base, CH)])
            pltpu.sync_copy(w_v, lw_hbm.at[2, pl.ds(base, CH)])
            return carry

        lax.fori_loop(0, N_GROUPS, group, 0)

    return gather_k


_EBLK = 3200


def _mlp_body(lw_ref, wf_ref, w0_ref, w1_ref, w2_ref, w3_ref, w4_ref,
              w5_ref, eye_ref, out_ref):
    locs = lw_ref[pl.ds(0, 2), :]
    w = lw_ref[pl.ds(2, 1), :]
    h = lax.dot_general(w0_ref[...], locs, (((1,), (0,)), ((), ())),
                        preferred_element_type=jnp.float32)
    h = _sin(h)
    for wref in (w1_ref, w2_ref, w3_ref, w4_ref):
        h = _sin(lax.dot_general(wref[...], h, (((1,), (0,)), ((), ())),
                                 preferred_element_type=jnp.float32))
    h5 = lax.dot_general(w5_ref[...], h, (((1,), (0,)), ((), ())),
                         preferred_element_type=jnp.float32)
    wft = lax.dot_general(eye_ref[...], wf_ref[...], (((1,), (1,)), ((), ())),
                          preferred_element_type=jnp.float32,
                          precision=lax.Precision.HIGHEST)
    wf4 = wft.reshape(4, 4, _EBLK)
    h4 = h5.reshape(4, 4, _EBLK)
    valt = jnp.sum(wf4[:, :, None, :] * h4[None, :, :, :], axis=1)
    valt = valt.reshape(16, _EBLK) * w
    vals = lax.dot_general(valt, eye_ref[...], (((0,), (0,)), ((), ())),
                           preferred_element_type=jnp.float32,
                           precision=lax.Precision.HIGHEST)
    out_ref[...] = vals


def _mlp(lw, wf, ws, eye):
    wspecs = [pl.BlockSpec(w.shape, lambda i: (0, 0)) for w in ws]
    return pl.pallas_call(
        _mlp_body,
        grid=(E_PAD // _EBLK,),
        in_specs=[
            pl.BlockSpec((3, _EBLK), lambda i: (0, i)),
            pl.BlockSpec((_EBLK, 16), lambda i: (i, 0)),
            *wspecs,
            pl.BlockSpec((16, 16), lambda i: (0, 0)),
        ],
        out_specs=pl.BlockSpec((_EBLK, 16), lambda i: (i, 0)),
        out_shape=jax.ShapeDtypeStruct((E_PAD, 16), jnp.float32),
    )(lw, wf, *ws, eye)


@functools.cache
def _make_scatter():
    mesh = plsc.VectorSubcoreMesh(core_axis_name="c", subcore_axis_name="s",
                                  num_cores=NC, num_subcores=NS)

    @functools.partial(
        pl.kernel,
        out_type=jax.ShapeDtypeStruct((NC, N_PTS, 16), jnp.float32),
        mesh=mesh,
        scratch_types=[
            pltpu.VMEM((G, 128), jnp.int32),
            pltpu.VMEM((CH, 16), jnp.float32),
            pltpu.VMEM((125, 16), jnp.float32),
            pltpu.VMEM_SHARED((N_ACC, 16), jnp.float32),
            pltpu.SemaphoreType.DMA,
        ],
        compiler_params=pltpu.CompilerParams(use_tc_tiling_on_sc=False),
    )
    def scatter_k(idxo_hbm, val_hbm, out_hbm, idxo_v, val_v, zb_v, acc_sh, sem):
        c = lax.axis_index("c")
        s = lax.axis_index("s")
        wid = s * NC + c

        zero16 = jnp.zeros((16,), jnp.float32)

        def zrow(i, carry):
            zb_v[i] = zero16
            return carry

        lax.fori_loop(0, 125, zrow, 0)

        def zacc(i, carry):
            pltpu.sync_copy(zb_v, acc_sh.at[pl.ds(s * ROWS_T + i * 125, 125)])
            return carry

        lax.fori_loop(0, 25, zacc, 0)

        @pl.when(s == 0)
        def _():
            pltpu.sync_copy(zb_v, acc_sh.at[pl.ds(N_ACC - 125, 125)])

        plsc.subcore_barrier()

        def group(g, carry):
            base = wid * E_W + g * CH
            pltpu.sync_copy(idxo_hbm.at[wid, pl.ds(g * G, G)], idxo_v)
            pltpu.sync_copy(val_hbm.at[pl.ds(base, CH)], val_v)
            for j in range(G):
                pltpu.sync_copy(val_v.at[pl.ds(j * 128, 128)],
                                acc_sh.at[idxo_v.at[j]], add=True)
            return carry

        lax.fori_loop(0, N_GROUPS, group, 0)

        plsc.subcore_barrier()

        pltpu.sync_copy(acc_sh.at[pl.ds(s * ROWS_T, ROWS_T)],
                        out_hbm.at[c, pl.ds(s * ROWS_T, ROWS_T)])

    return scatter_k


_FBLK = 2000


def _final_body(p_ref, eye_ref, o_ref):
    psum = p_ref[0] + p_ref[1]
    o_ref[...] = lax.dot_general(eye_ref[...], psum, (((1,), (1,)), ((), ())),
                                 preferred_element_type=jnp.float32,
                                 precision=lax.Precision.HIGHEST)


def _final(partials, eye):
    return pl.pallas_call(
        _final_body,
        out_shape=jax.ShapeDtypeStruct((16, N_PTS), jnp.float32),
    )(partials, eye)


def kernel(features, in_points, out_points, node_weights, eval_indices,
           W0, W1, W2, W3, W4, W5):
    n_extra = E_PAD - E
    pad_src = (jnp.arange(n_extra, dtype=jnp.int32) * 37) % N_PTS
    pad_dst = N_PTS + (jnp.arange(n_extra, dtype=jnp.int32) % (N_ACC - N_PTS))
    idx_out = jnp.concatenate([eval_indices[:, 0], pad_dst])
    idx_in = jnp.concatenate([eval_indices[:, 1], pad_src])
    idxo3 = idx_out.reshape(NW, ROWS_W, 128)
    idxi3 = idx_in.reshape(NW, ROWS_W, 128)

    feat_rows = features.reshape(16, N_PTS).T
    eye = jnp.eye(16, dtype=jnp.float32)
    planes = jnp.zeros((5, N_PLANE), jnp.float32)
    planes = planes.at[:, :N_PTS].set(jnp.stack([
        out_points[:, 0], out_points[:, 1],
        in_points[:, 0], in_points[:, 1], node_weights]))

    wf, lw = _make_gather()(idxo3, idxi3, feat_rows, planes)
    vals = _mlp(lw, wf, (W0, W1, W2, W3, W4, W5), eye)
    partials = _make_scatter()(idxo3, vals)
    out16 = _final(partials, eye)
    return out16.reshape(4, 4, N_PTS)


# speedup vs baseline: 81.6523x; 1.5252x over previous
import functools

import jax
import jax.numpy as jnp
from jax import lax
from jax.experimental import pallas as pl
from jax.experimental.pallas import tpu as pltpu
from jax.experimental.pallas import tpu_sc as plsc

N_PTS = 50000
E = 800000
NC, NS = 2, 16
NW = NC * NS
E_PAD = 819200
ROWS_W = 200
E_W = ROWS_W * 128
G = 8
N_GROUPS = ROWS_W // G
CH = G * 128
N_ACC = 50048
ROWS_T = N_PTS // NS

_SC = (0.9999999778617678, -0.16666645303901012, 0.008333121801711259,
       -0.00019833888177304604, 2.7436987965212285e-06,
       -2.404730690096224e-08, 1.1825182499869413e-10)


def _sin(x):
    z = x * x
    p = jnp.float32(_SC[6])
    for c in (_SC[5], _SC[4], _SC[3], _SC[2], _SC[1], _SC[0]):
        p = p * z + jnp.float32(c)
    return x * p


N_PLANE = 50048
ROWS_P = N_PLANE // NS
@functools.cache
def _make_gather():
    mesh = plsc.VectorSubcoreMesh(core_axis_name="c", subcore_axis_name="s",
                                  num_cores=NC, num_subcores=NS)

    @functools.partial(
        pl.kernel,
        out_type=(
            jax.ShapeDtypeStruct((E_PAD, 16), jnp.float32),
            jax.ShapeDtypeStruct((3, E_PAD), jnp.float32),
        ),
        mesh=mesh,
        scratch_types=[
            pltpu.VMEM((G, 128), jnp.int32),
            pltpu.VMEM((G, 128), jnp.int32),
            pltpu.VMEM((CH, 16), jnp.float32),
            pltpu.VMEM((CH,), jnp.float32),
            pltpu.VMEM((CH,), jnp.float32),
            pltpu.VMEM((CH,), jnp.float32),
            pltpu.VMEM((CH,), jnp.float32),
            pltpu.VMEM((CH,), jnp.float32),
            pltpu.VMEM_SHARED((N_PTS, 16), jnp.float32),
            pltpu.VMEM_SHARED((5, N_PLANE), jnp.float32),
            pltpu.SemaphoreType.DMA,
        ],
        compiler_params=pltpu.CompilerParams(use_tc_tiling_on_sc=False),
    )
    def gather_k(idxo_hbm, idxi_hbm, ft_hbm, pln_hbm,
                 wf_hbm, lw_hbm,
                 idxo_v, idxi_v, wf_v, ox_v, oy_v, ix_v, iy_v, w_v,
                 ft_sh, pln_sh, sem):
        s = lax.axis_index("s")
        wid = s * NC + lax.axis_index("c")

        descs = [pltpu.async_copy(ft_hbm.at[pl.ds(s * ROWS_T, ROWS_T)],
                                  ft_sh.at[pl.ds(s * ROWS_T, ROWS_T)], sem)]
        for p in range(5):
            descs.append(pltpu.async_copy(
                pln_hbm.at[p, pl.ds(s * ROWS_P, ROWS_P)],
                pln_sh.at[p, pl.ds(s * ROWS_P, ROWS_P)], sem))
        for d in descs:
            d.wait()
        plsc.subcore_barrier()

        def group(g, carry):
            base = wid * E_W + g * CH
            pltpu.sync_copy(idxo_hbm.at[wid, pl.ds(g * G, G)], idxo_v)
            pltpu.sync_copy(idxi_hbm.at[wid, pl.ds(g * G, G)], idxi_v)
            descs = []
            for j in range(G):
                dst = pl.ds(j * 128, 128)
                descs.append(pltpu.async_copy(
                    ft_sh.at[idxi_v.at[j]], wf_v.at[dst], sem))
                descs.append(pltpu.async_copy(
                    pln_sh.at[0].at[idxo_v.at[j]], ox_v.at[dst], sem))
                descs.append(pltpu.async_copy(
                    pln_sh.at[1].at[idxo_v.at[j]], oy_v.at[dst], sem))
                descs.append(pltpu.async_copy(
                    pln_sh.at[2].at[idxi_v.at[j]], ix_v.at[dst], sem))
                descs.append(pltpu.async_copy(
                    pln_sh.at[3].at[idxi_v.at[j]], iy_v.at[dst], sem))
                descs.append(pltpu.async_copy(
                    pln_sh.at[4].at[idxi_v.at[j]], w_v.at[dst], sem))
            for d in descs:
                d.wait()
            pltpu.sync_copy(wf_v, wf_hbm.at[pl.ds(base, CH)])
            for q in range(CH // 16):
                sl = pl.ds(q * 16, 16)
                ox_v[sl] = ox_v[sl] - ix_v[sl]
                oy_v[sl] = oy_v[sl] - iy_v[sl]
            pltpu.sync_copy(ox_v, lw_hbm.at[0, pl.ds(base, CH)])
            pltpu.sync_copy(oy_v, lw_hbm.at[1, pl.ds(base, CH)])
            pltpu.sync_copy(w_v, lw_hbm.at[2, pl.ds(base, CH)])
            return carry

        lax.fori_loop(0, N_GROUPS, group, 0)

    return gather_k


_EBLK = 6400


def _mlp_body(lw_ref, wf_ref, w0_ref, w1_ref, w2_ref, w3_ref, w4_ref,
              w5_ref, out_ref):
    locs = lw_ref[pl.ds(0, 2), :]
    w = lw_ref[pl.ds(2, 1), :]
    h = lax.dot_general(w0_ref[...], locs, (((1,), (0,)), ((), ())),
                        preferred_element_type=jnp.float32)
    h = _sin(h)
    for wref in (w1_ref, w2_ref, w3_ref, w4_ref):
        h = _sin(lax.dot_general(wref[...], h, (((1,), (0,)), ((), ())),
                                 preferred_element_type=jnp.float32))
    h5 = lax.dot_general(w5_ref[...], h, (((1,), (0,)), ((), ())),
                         preferred_element_type=jnp.float32)
    wft = wf_ref[...].T
    wf4 = wft.reshape(4, 4, _EBLK)
    h4 = h5.reshape(4, 4, _EBLK)
    valt = jnp.sum(wf4[:, :, None, :] * h4[None, :, :, :], axis=1)
    valt = valt.reshape(16, _EBLK) * w
    out_ref[...] = valt.T


def _mlp(lw, wf, ws):
    wspecs = [pl.BlockSpec(w.shape, lambda i: (0, 0)) for w in ws]
    return pl.pallas_call(
        _mlp_body,
        grid=(E_PAD // _EBLK,),
        in_specs=[
            pl.BlockSpec((3, _EBLK), lambda i: (0, i)),
            pl.BlockSpec((_EBLK, 16), lambda i: (i, 0)),
            *wspecs,
        ],
        out_specs=pl.BlockSpec((_EBLK, 16), lambda i: (i, 0)),
        out_shape=jax.ShapeDtypeStruct((E_PAD, 16), jnp.float32),
    )(lw, wf, *ws)


@functools.cache
def _make_scatter():
    mesh = plsc.VectorSubcoreMesh(core_axis_name="c", subcore_axis_name="s",
                                  num_cores=NC, num_subcores=NS)

    @functools.partial(
        pl.kernel,
        out_type=jax.ShapeDtypeStruct((NC, N_PTS, 16), jnp.float32),
        mesh=mesh,
        scratch_types=[
            pltpu.VMEM((G, 128), jnp.int32),
            pltpu.VMEM((CH, 16), jnp.float32),
            pltpu.VMEM((125, 16), jnp.float32),
            pltpu.VMEM_SHARED((N_ACC, 16), jnp.float32),
            pltpu.SemaphoreType.DMA,
        ],
        compiler_params=pltpu.CompilerParams(use_tc_tiling_on_sc=False),
    )
    def scatter_k(idxo_hbm, val_hbm, out_hbm, idxo_v, val_v, zb_v, acc_sh, sem):
        c = lax.axis_index("c")
        s = lax.axis_index("s")
        wid = s * NC + c

        zero16 = jnp.zeros((16,), jnp.float32)

        def zrow(i, carry):
            zb_v[i] = zero16
            return carry

        lax.fori_loop(0, 125, zrow, 0)

        def zacc(i, carry):
            pltpu.sync_copy(zb_v, acc_sh.at[pl.ds(s * ROWS_T + i * 125, 125)])
            return carry

        lax.fori_loop(0, 25, zacc, 0)

        @pl.when(s == 0)
        def _():
            pltpu.sync_copy(zb_v, acc_sh.at[pl.ds(N_ACC - 125, 125)])

        plsc.subcore_barrier()

        def group(g, carry):
            base = wid * E_W + g * CH
            pltpu.sync_copy(idxo_hbm.at[wid, pl.ds(g * G, G)], idxo_v)
            pltpu.sync_copy(val_hbm.at[pl.ds(base, CH)], val_v)
            for j in range(G):
                pltpu.sync_copy(val_v.at[pl.ds(j * 128, 128)],
                                acc_sh.at[idxo_v.at[j]], add=True)
            return carry

        lax.fori_loop(0, N_GROUPS, group, 0)

        plsc.subcore_barrier()

        pltpu.sync_copy(acc_sh.at[pl.ds(s * ROWS_T, ROWS_T)],
                        out_hbm.at[c, pl.ds(s * ROWS_T, ROWS_T)])

    return scatter_k


_FBLK = 2000


def _final_body(p_ref, o_ref):
    o_ref[...] = (p_ref[0] + p_ref[1]).T


def _final(partials):
    return pl.pallas_call(
        _final_body,
        out_shape=jax.ShapeDtypeStruct((16, N_PTS), jnp.float32),
    )(partials)


def kernel(features, in_points, out_points, node_weights, eval_indices,
           W0, W1, W2, W3, W4, W5):
    n_extra = E_PAD - E
    pad_src = (jnp.arange(n_extra, dtype=jnp.int32) * 37) % N_PTS
    pad_dst = N_PTS + (jnp.arange(n_extra, dtype=jnp.int32) % (N_ACC - N_PTS))
    idx_out = jnp.concatenate([eval_indices[:, 0], pad_dst])
    idx_in = jnp.concatenate([eval_indices[:, 1], pad_src])
    idxo3 = idx_out.reshape(NW, ROWS_W, 128)
    idxi3 = idx_in.reshape(NW, ROWS_W, 128)

    feat_rows = features.reshape(16, N_PTS).T
    planes = jnp.zeros((5, N_PLANE), jnp.float32)
    planes = planes.at[:, :N_PTS].set(jnp.stack([
        out_points[:, 0], out_points[:, 1],
        in_points[:, 0], in_points[:, 1], node_weights]))

    wf, lw = _make_gather()(idxo3, idxi3, feat_rows, planes)
    vals = _mlp(lw, wf, (W0, W1, W2, W3, W4, W5))
    partials = _make_scatter()(idxo3, vals)
    out16 = _final(partials)
    return out16.reshape(4, 4, N_PTS)
